# baseline (speedup 1.0000x reference)
import functools

import jax
import jax.numpy as jnp
from jax import lax
from jax.experimental import pallas as pl
from jax.experimental.pallas import tpu as pltpu
from jax.experimental.pallas import tpu_sc as plsc

N = 10000
E = 320000
D_IN = 128
H = 64
OUT = 16
G = 128
L = 5
N_PAD = 10112

NC, NS = 2, 16
HC = H // NC
CHUNK = 128
CHUNKS = 160
E_PAD = NS * CHUNKS * CHUNK
ROWS_PT = N_PAD // NS


def _bn(z, g, b):
    mu = jnp.mean(z, axis=0, keepdims=True)
    var = jnp.mean((z - mu) ** 2, axis=0, keepdims=True)
    return g * (z - mu) * lax.rsqrt(var + 1e-5) + b


def _split_out(out_ref, pn):
    out_ref[0, :N, :] = pn[:, :HC]
    out_ref[1, :N, :] = pn[:, HC:]
    out_ref[0, N:, :] = jnp.zeros((N_PAD - N, HC), jnp.float32)
    out_ref[1, N:, :] = jnp.zeros((N_PAD - N, HC), jnp.float32)


def _dense0_body(x_ref, w_ref, out_ref):
    _split_out(out_ref, jnp.dot(x_ref[:], w_ref[:],
                                preferred_element_type=jnp.float32, precision=lax.Precision.HIGHEST))


def _layer_body(agg_ref, b1_ref, g1_ref, be1_ref, w2_ref, b2_ref,
                gn_ref, bn_ref, w1n_ref, out_ref, *, last):
    z = jnp.concatenate([agg_ref[0, :N, :], agg_ref[1, :N, :]], axis=1)
    z = z + b1_ref[:]
    z = jnp.maximum(_bn(z, g1_ref[:], be1_ref[:]), 0.0)
    z = jnp.dot(z, w2_ref[:], preferred_element_type=jnp.float32, precision=lax.Precision.HIGHEST) + b2_ref[:]
    if last:
        out_ref[:N, :] = z
        out_ref[N:, :] = jnp.zeros((N_PAD - N, H), jnp.float32)
    else:
        h = jnp.maximum(_bn(z, gn_ref[:], bn_ref[:]), 0.0)
        _split_out(out_ref, jnp.dot(h, w1n_ref[:],
                                    preferred_element_type=jnp.float32, precision=lax.Precision.HIGHEST))


def _pool_body(h_ref, batch_ref, fc1w_ref, fc1b_ref, fc2w_ref, fc2b_ref,
               out_ref):
    seg = batch_ref[:]
    onehot = (seg == lax.broadcasted_iota(jnp.int32, (N, G), 1))
    onehot = onehot.astype(jnp.float32)
    sums = lax.dot_general(onehot, h_ref[:N, :],
                           (((0,), (0,)), ((), ())),
                           preferred_element_type=jnp.float32, precision=lax.Precision.HIGHEST)
    cnt = jnp.sum(onehot, axis=0, keepdims=True)
    pooled = sums / jnp.clip(cnt, 1.0, None).T
    y = jnp.maximum(
        jnp.dot(pooled, fc1w_ref[:], preferred_element_type=jnp.float32, precision=lax.Precision.HIGHEST)
        + fc1b_ref[:], 0.0)
    out_ref[:] = jnp.dot(y, fc2w_ref[:],
                         preferred_element_type=jnp.float32, precision=lax.Precision.HIGHEST) + fc2b_ref[:]


_dense0 = pl.pallas_call(
    _dense0_body,
    out_shape=jax.ShapeDtypeStruct((NC, N_PAD, HC), jnp.float32),
)

_pool = pl.pallas_call(
    _pool_body,
    out_shape=jax.ShapeDtypeStruct((G, OUT), jnp.float32),
)


def _agg_body(src_hbm, dst_hbm, p_hbm, out_hbm,
              srcv, dstv, rows,
              gs0, gs1, gs2, gs3, ss0, ss1, ss2, ss3, acc, p_sp):
    cid = lax.axis_index("c")
    sid = lax.axis_index("s")
    gsem = (gs0, gs1, gs2, gs3)
    ssem = (ss0, ss1, ss2, ss3)

    def gather(t, b):
        pltpu.async_copy(p_sp.at[srcv.at[t]], rows.at[b], gsem[b])

    def gwait(b):
        pltpu.make_async_copy(p_sp.at[srcv.at[0]], rows.at[b],
                              gsem[b]).wait()

    def scatter(t, b):
        pltpu.async_copy(rows.at[b], acc.at[dstv.at[t]], ssem[b], add=True)

    def swait(b):
        pltpu.make_async_copy(rows.at[b], acc.at[dstv.at[0]],
                              ssem[b]).wait()

    r0 = sid * ROWS_PT
    pltpu.sync_copy(p_hbm.at[cid, pl.ds(r0, ROWS_PT)],
                    p_sp.at[pl.ds(r0, ROWS_PT)])
    pltpu.sync_copy(p_hbm.at[cid, pl.ds(r0, ROWS_PT)],
                    acc.at[pl.ds(r0, ROWS_PT)])
    pltpu.sync_copy(src_hbm.at[sid], srcv)
    pltpu.sync_copy(dst_hbm.at[sid], dstv)
    plsc.subcore_barrier()

    gather(0, 0)
    gather(1, 1)
    gwait(0)
    scatter(0, 0)
    gather(2, 2)
    gwait(1)
    scatter(1, 1)
    gather(3, 3)

    def rnd(r, _):
        for j in range(4):
            t = 4 * r + 2 + j
            b = (2 + j) % 4
            gwait(b)
            scatter(t, b)
            swait((b + 2) % 4)
            gather(t + 2, (b + 2) % 4)
        return 0

    lax.fori_loop(0, (CHUNKS - 4) // 4, rnd, 0)
    gwait(2)
    scatter(CHUNKS - 2, 2)
    swait(0)
    gwait(3)
    scatter(CHUNKS - 1, 3)
    swait(1)
    swait(2)
    swait(3)

    plsc.subcore_barrier()
    pltpu.sync_copy(acc.at[pl.ds(r0, ROWS_PT)],
                    out_hbm.at[cid, pl.ds(r0, ROWS_PT), :])


_agg_sc = functools.partial(
    pl.kernel,
    out_type=jax.ShapeDtypeStruct((NC, N_PAD, HC), jnp.float32),
    mesh=plsc.VectorSubcoreMesh(core_axis_name="c", subcore_axis_name="s"),
    scratch_types=[
        pltpu.VMEM((CHUNKS, CHUNK), jnp.int32),
        pltpu.VMEM((CHUNKS, CHUNK), jnp.int32),
        pltpu.VMEM((4, CHUNK, HC), jnp.float32),
        pltpu.SemaphoreType.DMA,
        pltpu.SemaphoreType.DMA,
        pltpu.SemaphoreType.DMA,
        pltpu.SemaphoreType.DMA,
        pltpu.SemaphoreType.DMA,
        pltpu.SemaphoreType.DMA,
        pltpu.SemaphoreType.DMA,
        pltpu.SemaphoreType.DMA,
        pltpu.VMEM_SHARED((N_PAD, HC), jnp.float32),
        pltpu.VMEM_SHARED((N_PAD, HC), jnp.float32),
    ],
    compiler_params=pltpu.CompilerParams(use_tc_tiling_on_sc=False),
)(_agg_body)


def _agg_edges(p_split, src_t, dst_t):
    return _agg_sc(src_t, dst_t, p_split)


def kernel(x, edge_index, batch, W1_0, W1_r, b1, g1, be1, W2, b2, gn, bn,
           fc1_W, fc1_b, fc2_W, fc2_b):
    pad = N + (jnp.arange(E_PAD - E, dtype=jnp.int32) % (N_PAD - N))
    src_t = jnp.concatenate([edge_index[0], pad]).reshape(NS, CHUNKS, CHUNK)
    dst_t = jnp.concatenate([edge_index[1], pad]).reshape(NS, CHUNKS, CHUNK)

    p = _dense0(x, W1_0)
    for i in range(L):
        agg = _agg_edges(p, src_t, dst_t)
        last = i == L - 1
        w1n = W1_r[i] if not last else jnp.zeros((H, H), jnp.float32)
        layer = pl.pallas_call(
            functools.partial(_layer_body, last=last),
            out_shape=jax.ShapeDtypeStruct(
                (N_PAD, H) if last else (NC, N_PAD, HC), jnp.float32),
        )
        p = layer(agg, b1[i].reshape(1, H), g1[i].reshape(1, H),
                  be1[i].reshape(1, H), W2[i], b2[i].reshape(1, H),
                  (gn[i] if not last else gn[0]).reshape(1, H),
                  (bn[i] if not last else bn[0]).reshape(1, H), w1n)

    return _pool(p, batch.reshape(N, 1), fc1_W, fc1_b.reshape(1, H),
                 fc2_W, fc2_b.reshape(1, OUT))


# speedup vs baseline: 1.0052x; 1.0052x over previous
import functools

import jax
import jax.numpy as jnp
from jax import lax
from jax.experimental import pallas as pl
from jax.experimental.pallas import tpu as pltpu
from jax.experimental.pallas import tpu_sc as plsc

N = 10000
E = 320000
D_IN = 128
H = 64
OUT = 16
G = 128
L = 5
N_PAD = 10112

NC, NS = 2, 16
HC = H // NC
CHUNK = 128
CHUNKS = 160
E_PAD = NS * CHUNKS * CHUNK
ROWS_PT = N_PAD // NS


def _bn(z, g, b):
    mu = jnp.mean(z, axis=0, keepdims=True)
    var = jnp.mean((z - mu) ** 2, axis=0, keepdims=True)
    return g * (z - mu) * lax.rsqrt(var + 1e-5) + b


def _split_out(out_ref, pn):
    out_ref[0, :N, :] = pn[:, :HC]
    out_ref[1, :N, :] = pn[:, HC:]
    out_ref[0, N:, :] = jnp.zeros((N_PAD - N, HC), jnp.float32)
    out_ref[1, N:, :] = jnp.zeros((N_PAD - N, HC), jnp.float32)


def _dense0_body(x_ref, w_ref, out_ref):
    _split_out(out_ref, jnp.dot(x_ref[:], w_ref[:],
                                preferred_element_type=jnp.float32, precision=lax.Precision.HIGHEST))


def _layer_body(agg_ref, b1_ref, g1_ref, be1_ref, w2_ref, b2_ref,
                gn_ref, bn_ref, w1n_ref, out_ref, *, last):
    z = jnp.concatenate([agg_ref[0, :N, :], agg_ref[1, :N, :]], axis=1)
    z = z + b1_ref[:]
    z = jnp.maximum(_bn(z, g1_ref[:], be1_ref[:]), 0.0)
    z = jnp.dot(z, w2_ref[:], preferred_element_type=jnp.float32, precision=lax.Precision.HIGHEST) + b2_ref[:]
    if last:
        out_ref[:N, :] = z
        out_ref[N:, :] = jnp.zeros((N_PAD - N, H), jnp.float32)
    else:
        h = jnp.maximum(_bn(z, gn_ref[:], bn_ref[:]), 0.0)
        _split_out(out_ref, jnp.dot(h, w1n_ref[:],
                                    preferred_element_type=jnp.float32, precision=lax.Precision.HIGHEST))


def _pool_body(h_ref, batch_ref, fc1w_ref, fc1b_ref, fc2w_ref, fc2b_ref,
               out_ref):
    seg = batch_ref[:]
    onehot = (seg == lax.broadcasted_iota(jnp.int32, (N, G), 1))
    onehot = onehot.astype(jnp.float32)
    sums = lax.dot_general(onehot, h_ref[:N, :],
                           (((0,), (0,)), ((), ())),
                           preferred_element_type=jnp.float32)
    cnt = jnp.sum(onehot, axis=0, keepdims=True)
    pooled = sums / jnp.clip(cnt, 1.0, None).T
    y = jnp.maximum(
        jnp.dot(pooled, fc1w_ref[:], preferred_element_type=jnp.float32)
        + fc1b_ref[:], 0.0)
    out_ref[:] = jnp.dot(y, fc2w_ref[:],
                         preferred_element_type=jnp.float32) + fc2b_ref[:]


_dense0 = pl.pallas_call(
    _dense0_body,
    out_shape=jax.ShapeDtypeStruct((NC, N_PAD, HC), jnp.float32),
)

_pool = pl.pallas_call(
    _pool_body,
    out_shape=jax.ShapeDtypeStruct((G, OUT), jnp.float32),
)


def _agg_body(src_hbm, dst_hbm, p_hbm, out_hbm,
              srcv, dstv, rows,
              gs0, gs1, gs2, gs3, ss0, ss1, ss2, ss3, acc, p_sp):
    cid = lax.axis_index("c")
    sid = lax.axis_index("s")
    gsem = (gs0, gs1, gs2, gs3)
    ssem = (ss0, ss1, ss2, ss3)

    def gather(t, b):
        pltpu.async_copy(p_sp.at[srcv.at[t]], rows.at[b], gsem[b])

    def gwait(b):
        pltpu.make_async_copy(p_sp.at[srcv.at[0]], rows.at[b],
                              gsem[b]).wait()

    def scatter(t, b):
        pltpu.async_copy(rows.at[b], acc.at[dstv.at[t]], ssem[b], add=True)

    def swait(b):
        pltpu.make_async_copy(rows.at[b], acc.at[dstv.at[0]],
                              ssem[b]).wait()

    r0 = sid * ROWS_PT
    pltpu.sync_copy(p_hbm.at[cid, pl.ds(r0, ROWS_PT)],
                    p_sp.at[pl.ds(r0, ROWS_PT)])
    pltpu.sync_copy(p_hbm.at[cid, pl.ds(r0, ROWS_PT)],
                    acc.at[pl.ds(r0, ROWS_PT)])
    pltpu.sync_copy(src_hbm.at[sid], srcv)
    pltpu.sync_copy(dst_hbm.at[sid], dstv)
    plsc.subcore_barrier()

    gather(0, 0)
    gather(1, 1)
    gwait(0)
    scatter(0, 0)
    gather(2, 2)
    gwait(1)
    scatter(1, 1)
    gather(3, 3)

    def rnd(r, _):
        for j in range(4):
            t = 4 * r + 2 + j
            b = (2 + j) % 4
            gwait(b)
            scatter(t, b)
            swait((b + 2) % 4)
            gather(t + 2, (b + 2) % 4)
        return 0

    lax.fori_loop(0, (CHUNKS - 4) // 4, rnd, 0)
    gwait(2)
    scatter(CHUNKS - 2, 2)
    swait(0)
    gwait(3)
    scatter(CHUNKS - 1, 3)
    swait(1)
    swait(2)
    swait(3)

    plsc.subcore_barrier()
    pltpu.sync_copy(acc.at[pl.ds(r0, ROWS_PT)],
                    out_hbm.at[cid, pl.ds(r0, ROWS_PT), :])


_agg_sc = functools.partial(
    pl.kernel,
    out_type=jax.ShapeDtypeStruct((NC, N_PAD, HC), jnp.float32),
    mesh=plsc.VectorSubcoreMesh(core_axis_name="c", subcore_axis_name="s"),
    scratch_types=[
        pltpu.VMEM((CHUNKS, CHUNK), jnp.int32),
        pltpu.VMEM((CHUNKS, CHUNK), jnp.int32),
        pltpu.VMEM((4, CHUNK, HC), jnp.float32),
        pltpu.SemaphoreType.DMA,
        pltpu.SemaphoreType.DMA,
        pltpu.SemaphoreType.DMA,
        pltpu.SemaphoreType.DMA,
        pltpu.SemaphoreType.DMA,
        pltpu.SemaphoreType.DMA,
        pltpu.SemaphoreType.DMA,
        pltpu.SemaphoreType.DMA,
        pltpu.VMEM_SHARED((N_PAD, HC), jnp.float32),
        pltpu.VMEM_SHARED((N_PAD, HC), jnp.float32),
    ],
    compiler_params=pltpu.CompilerParams(use_tc_tiling_on_sc=False),
)(_agg_body)


def _agg_edges(p_split, src_t, dst_t):
    return _agg_sc(src_t, dst_t, p_split)


def kernel(x, edge_index, batch, W1_0, W1_r, b1, g1, be1, W2, b2, gn, bn,
           fc1_W, fc1_b, fc2_W, fc2_b):
    pad = N + (jnp.arange(E_PAD - E, dtype=jnp.int32) % (N_PAD - N))
    src_t = jnp.concatenate([edge_index[0], pad]).reshape(NS, CHUNKS, CHUNK)
    dst_t = jnp.concatenate([edge_index[1], pad]).reshape(NS, CHUNKS, CHUNK)

    p = _dense0(x, W1_0)
    for i in range(L):
        agg = _agg_edges(p, src_t, dst_t)
        last = i == L - 1
        w1n = W1_r[i] if not last else jnp.zeros((H, H), jnp.float32)
        layer = pl.pallas_call(
            functools.partial(_layer_body, last=last),
            out_shape=jax.ShapeDtypeStruct(
                (N_PAD, H) if last else (NC, N_PAD, HC), jnp.float32),
        )
        p = layer(agg, b1[i].reshape(1, H), g1[i].reshape(1, H),
                  be1[i].reshape(1, H), W2[i], b2[i].reshape(1, H),
                  (gn[i] if not last else gn[0]).reshape(1, H),
                  (bn[i] if not last else bn[0]).reshape(1, H), w1n)

    return _pool(p, batch.reshape(N, 1), fc1_W, fc1_b.reshape(1, H),
                 fc2_W, fc2_b.reshape(1, OUT))


# speedup vs baseline: 1.1332x; 1.1273x over previous
import functools

import jax
import jax.numpy as jnp
from jax import lax
from jax.experimental import pallas as pl
from jax.experimental.pallas import tpu as pltpu
from jax.experimental.pallas import tpu_sc as plsc

N = 10000
E = 320000
D_IN = 128
H = 64
OUT = 16
G = 128
L = 5
N_PAD = 10112

NC, NS = 2, 16
HC = H // NC
CHUNK = 128
CHUNKS = 160
E_PAD = NS * CHUNKS * CHUNK
ROWS_PT = N_PAD // NS


def _bn(z, g, b):
    mu = jnp.mean(z, axis=0, keepdims=True)
    var = jnp.mean((z - mu) ** 2, axis=0, keepdims=True)
    return g * (z - mu) * lax.rsqrt(var + 1e-5) + b


def _split_out(out_ref, pn):
    out_ref[:N, :] = jnp.concatenate(
        [pn, jnp.zeros((N, 128 - H), jnp.float32)], axis=1)
    out_ref[N:, :] = jnp.zeros((N_PAD - N, 128), jnp.float32)


def _dense0_body(x_ref, w_ref, out_ref):
    _split_out(out_ref, jnp.dot(x_ref[:], w_ref[:],
                                preferred_element_type=jnp.float32, precision=lax.Precision.HIGHEST))


def _layer_body(agg_ref, b1_ref, g1_ref, be1_ref, w2_ref, b2_ref,
                gn_ref, bn_ref, w1n_ref, out_ref, *, last):
    z = agg_ref[:N, :H] + b1_ref[:]
    z = jnp.maximum(_bn(z, g1_ref[:], be1_ref[:]), 0.0)
    z = jnp.dot(z, w2_ref[:], preferred_element_type=jnp.float32, precision=lax.Precision.HIGHEST) + b2_ref[:]
    if last:
        out_ref[:N, :] = z
        out_ref[N:, :] = jnp.zeros((N_PAD - N, H), jnp.float32)
    else:
        h = jnp.maximum(_bn(z, gn_ref[:], bn_ref[:]), 0.0)
        _split_out(out_ref, jnp.dot(h, w1n_ref[:],
                                    preferred_element_type=jnp.float32, precision=lax.Precision.HIGHEST))


def _pool_body(h_ref, batch_ref, fc1w_ref, fc1b_ref, fc2w_ref, fc2b_ref,
               out_ref):
    seg = batch_ref[:]
    onehot = (seg == lax.broadcasted_iota(jnp.int32, (N, G), 1))
    onehot = onehot.astype(jnp.float32)
    sums = lax.dot_general(onehot, h_ref[:N, :],
                           (((0,), (0,)), ((), ())),
                           preferred_element_type=jnp.float32)
    cnt = jnp.sum(onehot, axis=0, keepdims=True)
    pooled = sums / jnp.clip(cnt, 1.0, None).T
    y = jnp.maximum(
        jnp.dot(pooled, fc1w_ref[:], preferred_element_type=jnp.float32)
        + fc1b_ref[:], 0.0)
    out_ref[:] = jnp.dot(y, fc2w_ref[:],
                         preferred_element_type=jnp.float32) + fc2b_ref[:]


_dense0 = pl.pallas_call(
    _dense0_body,
    out_shape=jax.ShapeDtypeStruct((N_PAD, 128), jnp.float32),
)

_pool = pl.pallas_call(
    _pool_body,
    out_shape=jax.ShapeDtypeStruct((G, OUT), jnp.float32),
)


def _agg_body(src_hbm, dst_hbm, p_hbm, out_hbm,
              srcv, dstv, rows,
              gs0, gs1, gs2, gs3, ss0, ss1, ss2, ss3, acc, p_sp):
    cid = lax.axis_index("c")
    sid = lax.axis_index("s")
    gsem = (gs0, gs1, gs2, gs3)
    ssem = (ss0, ss1, ss2, ss3)

    def gather(t, b):
        pltpu.async_copy(p_sp.at[srcv.at[t]], rows.at[b], gsem[b])

    def gwait(b):
        pltpu.make_async_copy(p_sp.at[srcv.at[0]], rows.at[b],
                              gsem[b]).wait()

    def scatter(t, b):
        pltpu.async_copy(rows.at[b], acc.at[dstv.at[t]], ssem[b], add=True)

    def swait(b):
        pltpu.make_async_copy(rows.at[b], acc.at[dstv.at[0]],
                              ssem[b]).wait()

    r0 = sid * ROWS_PT
    c0 = cid * HC
    pltpu.sync_copy(p_hbm.at[pl.ds(r0, ROWS_PT), pl.ds(c0, HC)],
                    p_sp.at[pl.ds(r0, ROWS_PT)])
    pltpu.sync_copy(p_hbm.at[pl.ds(r0, ROWS_PT), pl.ds(c0, HC)],
                    acc.at[pl.ds(r0, ROWS_PT)])
    pltpu.sync_copy(src_hbm.at[sid], srcv)
    pltpu.sync_copy(dst_hbm.at[sid], dstv)
    plsc.subcore_barrier()

    gather(0, 0)
    gather(1, 1)
    gwait(0)
    scatter(0, 0)
    gather(2, 2)
    gwait(1)
    scatter(1, 1)
    gather(3, 3)

    def rnd(r, _):
        for j in range(4):
            t = 4 * r + 2 + j
            b = (2 + j) % 4
            gwait(b)
            scatter(t, b)
            swait((b + 2) % 4)
            gather(t + 2, (b + 2) % 4)
        return 0

    lax.fori_loop(0, (CHUNKS - 4) // 4, rnd, 0)
    gwait(2)
    scatter(CHUNKS - 2, 2)
    swait(0)
    gwait(3)
    scatter(CHUNKS - 1, 3)
    swait(1)
    swait(2)
    swait(3)

    plsc.subcore_barrier()
    pltpu.sync_copy(acc.at[pl.ds(r0, ROWS_PT)],
                    out_hbm.at[pl.ds(r0, ROWS_PT), pl.ds(c0, HC)])


_agg_sc = functools.partial(
    pl.kernel,
    out_type=jax.ShapeDtypeStruct((N_PAD, 128), jnp.float32),
    mesh=plsc.VectorSubcoreMesh(core_axis_name="c", subcore_axis_name="s"),
    scratch_types=[
        pltpu.VMEM((CHUNKS, CHUNK), jnp.int32),
        pltpu.VMEM((CHUNKS, CHUNK), jnp.int32),
        pltpu.VMEM((4, CHUNK, HC), jnp.float32),
        pltpu.SemaphoreType.DMA,
        pltpu.SemaphoreType.DMA,
        pltpu.SemaphoreType.DMA,
        pltpu.SemaphoreType.DMA,
        pltpu.SemaphoreType.DMA,
        pltpu.SemaphoreType.DMA,
        pltpu.SemaphoreType.DMA,
        pltpu.SemaphoreType.DMA,
        pltpu.VMEM_SHARED((N_PAD, HC), jnp.float32),
        pltpu.VMEM_SHARED((N_PAD, HC), jnp.float32),
    ],
    compiler_params=pltpu.CompilerParams(use_tc_tiling_on_sc=False),
)(_agg_body)


def _agg_edges(p_io, src_t, dst_t):
    return _agg_sc(src_t, dst_t, p_io)


def kernel(x, edge_index, batch, W1_0, W1_r, b1, g1, be1, W2, b2, gn, bn,
           fc1_W, fc1_b, fc2_W, fc2_b):
    pad = N + (jnp.arange(E_PAD - E, dtype=jnp.int32) % (N_PAD - N))
    src_t = jnp.concatenate([edge_index[0], pad]).reshape(NS, CHUNKS, CHUNK)
    dst_t = jnp.concatenate([edge_index[1], pad]).reshape(NS, CHUNKS, CHUNK)

    p = _dense0(x, W1_0)
    for i in range(L):
        agg = _agg_edges(p, src_t, dst_t)
        last = i == L - 1
        w1n = W1_r[i] if not last else jnp.zeros((H, H), jnp.float32)
        layer = pl.pallas_call(
            functools.partial(_layer_body, last=last),
            out_shape=jax.ShapeDtypeStruct(
                (N_PAD, H) if last else (N_PAD, 128), jnp.float32),
        )
        p = layer(agg, b1[i].reshape(1, H), g1[i].reshape(1, H),
                  be1[i].reshape(1, H), W2[i], b2[i].reshape(1, H),
                  (gn[i] if not last else gn[0]).reshape(1, H),
                  (bn[i] if not last else bn[0]).reshape(1, H), w1n)

    return _pool(p, batch.reshape(N, 1), fc1_W, fc1_b.reshape(1, H),
                 fc2_W, fc2_b.reshape(1, OUT))
